# Initial kernel scaffold; baseline (speedup 1.0000x reference)
import jax
import jax.numpy as jnp
from jax.experimental import pallas as pl


def kernel(x, bigram_ids, layer_id, collect_stats, E, W):
    raise NotImplementedError("write your pallas kernel here")


# speedup vs baseline: 3.1991x; 3.1991x over previous
import functools

import jax
import jax.numpy as jnp
from jax import lax
from jax.experimental import pallas as pl
from jax.experimental.pallas import tpu as pltpu
from jax.experimental.pallas import tpu_sc as plsc

TABLE_SIZE = 1000000
MEM_DIM = 128
DIM = 2048

_NC = 2
_NS = 16
_NW = _NC * _NS


def _sc_gather(table, idx, n_rows):
    b_per_w = n_rows // _NW
    mesh = plsc.VectorSubcoreMesh(core_axis_name="c", subcore_axis_name="s")

    @functools.partial(
        pl.kernel,
        mesh=mesh,
        out_type=jax.ShapeDtypeStruct((n_rows, MEM_DIM), jnp.float32),
        scratch_types=[
            pltpu.VMEM((b_per_w,), jnp.int32),
            pltpu.VMEM((b_per_w, MEM_DIM), jnp.float32),
            pltpu.SemaphoreType.DMA,
        ],
    )
    def gather_kernel(table_hbm, idx_hbm, out_hbm, idx_v, rows_v, sem):
        wid = lax.axis_index("s") * _NC + lax.axis_index("c")
        base = wid * b_per_w
        pltpu.sync_copy(idx_hbm.at[pl.ds(base, b_per_w)], idx_v)
        pltpu.async_copy(table_hbm.at[idx_v], rows_v, sem).wait()
        pltpu.sync_copy(rows_v, out_hbm.at[pl.ds(base, b_per_w)])

    return gather_kernel(table, idx)


def _tc_matmul_kernel(ids_ref, mem_ref, w_ref, out_ref):
    mask = (ids_ref[0, 0, :] != 0).astype(jnp.float32)
    mem = mem_ref[...] * mask[:, None]
    out_ref[...] = lax.dot_general(
        mem, w_ref[...], (((1,), (1,)), ((), ())),
        preferred_element_type=jnp.float32)


def _tc_matmul(mem, w, ids3, n_rows, block_rows):
    grid = (n_rows // block_rows,)
    return pl.pallas_call(
        _tc_matmul_kernel,
        grid=grid,
        in_specs=[
            pl.BlockSpec((1, 1, block_rows), lambda i: (i, 0, 0)),
            pl.BlockSpec((block_rows, MEM_DIM), lambda i: (i, 0)),
            pl.BlockSpec((DIM, MEM_DIM), lambda i: (0, 0)),
        ],
        out_specs=pl.BlockSpec((block_rows, DIM), lambda i: (i, 0)),
        out_shape=jax.ShapeDtypeStruct((n_rows, DIM), jnp.float32),
    )(ids3, mem, w)


def kernel(x, bigram_ids, layer_id, collect_stats, E, W):
    b, s = bigram_ids.shape
    n_rows = b * s
    ids = bigram_ids.reshape(n_rows).astype(jnp.int32)
    mem = _sc_gather(E, ids, n_rows)
    block_rows = 512
    ids3 = ids.reshape(n_rows // block_rows, 1, block_rows)
    out = _tc_matmul(mem, W, ids3, n_rows, block_rows)
    return out.reshape(b, s, DIM)


# speedup vs baseline: 3.5435x; 1.1076x over previous
import functools

import jax
import jax.numpy as jnp
from jax import lax
from jax.experimental import pallas as pl
from jax.experimental.pallas import tpu as pltpu
from jax.experimental.pallas import tpu_sc as plsc

TABLE_SIZE = 1000000
MEM_DIM = 128
DIM = 2048

_NC = 2
_NS = 16
_NW = _NC * _NS


def _sc_gather(table, idx, n_rows):
    b_per_w = n_rows // _NW
    mesh = plsc.VectorSubcoreMesh(core_axis_name="c", subcore_axis_name="s")

    @functools.partial(
        pl.kernel,
        mesh=mesh,
        out_type=jax.ShapeDtypeStruct((n_rows, MEM_DIM), jnp.float32),
        scratch_types=[
            pltpu.VMEM((b_per_w,), jnp.int32),
            pltpu.VMEM((b_per_w, MEM_DIM), jnp.float32),
            pltpu.SemaphoreType.DMA,
        ],
    )
    def gather_kernel(table_hbm, idx_hbm, out_hbm, idx_v, rows_v, sem):
        wid = lax.axis_index("s") * _NC + lax.axis_index("c")
        base = wid * b_per_w
        pltpu.sync_copy(idx_hbm.at[pl.ds(base, b_per_w)], idx_v)
        pltpu.async_copy(table_hbm.at[idx_v], rows_v, sem).wait()
        pltpu.sync_copy(rows_v, out_hbm.at[pl.ds(base, b_per_w)])

    return gather_kernel(table, idx)


def _tc_matmul_kernel(ids_ref, mem_ref, w_ref, out_ref):
    mask = (ids_ref[0, 0, :] != 0).astype(jnp.float32)
    mem = mem_ref[...] * mask[:, None]
    out_ref[...] = lax.dot_general(
        mem, w_ref[...], (((1,), (1,)), ((), ())),
        preferred_element_type=jnp.float32)


def _tc_matmul(mem, w, ids3, n_rows, block_rows):
    grid = (n_rows // block_rows,)
    return pl.pallas_call(
        _tc_matmul_kernel,
        grid=grid,
        in_specs=[
            pl.BlockSpec((1, 1, block_rows), lambda i: (i, 0, 0)),
            pl.BlockSpec((block_rows, MEM_DIM), lambda i: (i, 0)),
            pl.BlockSpec((DIM, MEM_DIM), lambda i: (0, 0)),
        ],
        out_specs=pl.BlockSpec((block_rows, DIM), lambda i: (i, 0)),
        out_shape=jax.ShapeDtypeStruct((n_rows, DIM), jnp.float32),
    )(ids3, mem, w)


def kernel(x, bigram_ids, layer_id, collect_stats, E, W):
    b, s = bigram_ids.shape
    n_rows = b * s
    ids = bigram_ids.reshape(n_rows).astype(jnp.int32)
    mem = _sc_gather(E, ids, n_rows)
    block_rows = 1024
    ids3 = ids.reshape(n_rows // block_rows, 1, block_rows)
    out = _tc_matmul(mem, W, ids3, n_rows, block_rows)
    return out.reshape(b, s, DIM)
